# baseline (speedup 1.0000x reference)
import functools

import jax
import jax.numpy as jnp
from jax import lax
from jax.experimental import pallas as pl
from jax.experimental.pallas import tpu as pltpu
from jax.experimental.pallas import tpu_sc as plsc

_NC = 2
_NS = 16
_LANES = 16


def _stats_body(x_ref, s_ref):
    @pl.when(pl.program_id(0) == 0)
    def _():
        s_ref[...] = jnp.zeros_like(s_ref)

    xb = x_ref[...]
    s0 = jnp.sum(xb, axis=0)
    s1 = jnp.sum(xb * xb, axis=0)
    s_ref[...] += jnp.stack([s0, s1])


def _bn_stats(x, nblk):
    n, f = x.shape
    rows = n // nblk
    return pl.pallas_call(
        _stats_body,
        grid=(nblk,),
        in_specs=[pl.BlockSpec((rows, f), lambda i: (i, 0))],
        out_specs=pl.BlockSpec((2, f), lambda i: (0, 0)),
        out_shape=jax.ShapeDtypeStruct((2, f), jnp.float32),
    )(x)


def _norm_silu_body(n_rows, x_ref, s_ref, g_ref, b_ref, h_ref):
    s = s_ref[...]
    mean = s[0] / n_rows
    var = s[1] / n_rows - mean * mean
    scale = g_ref[0] * lax.rsqrt(var + 1e-5)
    shift = b_ref[0] - mean * scale
    t = x_ref[...] * scale + shift
    h_ref[...] = t * jax.nn.sigmoid(t)


def _norm_silu(x, sums, gamma, beta, nblk):
    n, f = x.shape
    rows = n // nblk
    return pl.pallas_call(
        functools.partial(_norm_silu_body, float(n)),
        grid=(nblk,),
        in_specs=[
            pl.BlockSpec((rows, f), lambda i: (i, 0)),
            pl.BlockSpec((2, f), lambda i: (0, 0)),
            pl.BlockSpec((1, f), lambda i: (0, 0)),
            pl.BlockSpec((1, f), lambda i: (0, 0)),
        ],
        out_specs=pl.BlockSpec((rows, f), lambda i: (i, 0)),
        out_shape=jax.ShapeDtypeStruct((n, f), jnp.float32),
    )(x, sums, gamma.reshape(1, f), beta.reshape(1, f))


def _sc_gather(h, in_pad, ep):
    n, f = h.shape
    chunk = 128
    nbuf = 6
    per_w = ep // (_NC * _NS)
    iters = per_w // chunk
    mesh = plsc.VectorSubcoreMesh(core_axis_name="c", subcore_axis_name="s")

    @functools.partial(
        pl.kernel,
        mesh=mesh,
        out_type=jax.ShapeDtypeStruct((ep, f), jnp.float32),
        scratch_types=[
            pltpu.VMEM((per_w,), jnp.int32),
            *[pltpu.VMEM((chunk, f), jnp.float32) for _ in range(nbuf)],
            *[pltpu.SemaphoreType.DMA for _ in range(2 * nbuf)],
        ],
    )
    def gather_k(h_hbm, idx_hbm, out_hbm, idx_all, *bufs_sems):
        rows = bufs_sems[:nbuf]
        gsem = bufs_sems[nbuf:2 * nbuf]
        ssem = bufs_sems[2 * nbuf:]
        wid = lax.axis_index("s") * _NC + lax.axis_index("c")
        base = wid * per_w

        pltpu.sync_copy(idx_hbm.at[pl.ds(base, per_w)], idx_all)

        depth = nbuf - 1
        ghandles = [None] * nbuf
        shandles = [None] * nbuf

        def issue_gather(j):
            p = j % nbuf
            ghandles[p] = pltpu.async_copy(
                h_hbm.at[idx_all.at[pl.ds(j * chunk, chunk)]],
                rows[p], gsem[p])

        for j in range(min(depth, iters)):
            issue_gather(j)
        for i in range(iters):
            p = i % nbuf
            ghandles[p].wait()
            shandles[p] = pltpu.async_copy(
                rows[p], out_hbm.at[pl.ds(base + i * chunk, chunk)], ssem[p])
            j = i + depth
            if j < iters:
                pj = j % nbuf
                if shandles[pj] is not None:
                    shandles[pj].wait()
                issue_gather(j)
        for p in range(nbuf):
            if shandles[p] is not None:
                shandles[p].wait()

    return gather_k(h, in_pad)


def _mm_body(bk_ref, x_ref, w_ref, y_ref):
    k = bk_ref[pl.program_id(0)]
    y_ref[...] = jnp.dot(x_ref[...], w_ref[k],
                         preferred_element_type=jnp.float32)


def _grouped_matmul(h_src, w, block_k, blk):
    ep, f = h_src.shape
    fout = w.shape[-1]
    nb = ep // blk
    kvol = w.shape[0]
    grid_spec = pltpu.PrefetchScalarGridSpec(
        num_scalar_prefetch=1,
        grid=(nb,),
        in_specs=[
            pl.BlockSpec((blk, f), lambda b, bk: (b, 0)),
            pl.BlockSpec((kvol, f, fout), lambda b, bk: (0, 0, 0)),
        ],
        out_specs=pl.BlockSpec((blk, fout), lambda b, bk: (b, 0)),
    )
    return pl.pallas_call(
        _mm_body,
        grid_spec=grid_spec,
        out_shape=jax.ShapeDtypeStruct((ep, fout), jnp.float32),
    )(block_k, h_src, w)


def _sc_scatter_add(y, out_pad, n_pad):
    ep, f = y.shape
    chunk = 32
    nranges = 4
    nr = n_pad // nranges
    region = nr + 8
    dump = nr
    lch = 512
    nch = nr // lch
    rem = ep - n_pad
    per_s = rem // _NS
    iters = per_s // chunk
    nbuf = 4
    outer = iters // nbuf
    mesh = plsc.VectorSubcoreMesh(core_axis_name="c", subcore_axis_name="s")

    @functools.partial(
        pl.kernel,
        mesh=mesh,
        out_type=jax.ShapeDtypeStruct((n_pad, f), jnp.float32),
        scratch_types=[
            pltpu.VMEM((nbuf, chunk), jnp.int32),
            pltpu.VMEM((nbuf, chunk), jnp.int32),
            pltpu.VMEM_SHARED((region, f), jnp.float32),
            *[pltpu.VMEM((chunk, f), jnp.float32) for _ in range(nbuf)],
            *[pltpu.SemaphoreType.DMA for _ in range(3 * nbuf)],
        ],
    )
    def scatter_k(y_hbm, opad_hbm, out_hbm, idx_raw, idx_loc, shared,
                  *bufs_sems):
        rows = bufs_sems[:nbuf]
        lsem = bufs_sems[nbuf:2 * nbuf]
        isem = bufs_sems[2 * nbuf:3 * nbuf]
        asem = bufs_sems[3 * nbuf:]
        c = lax.axis_index("c")
        s = lax.axis_index("s")
        base = n_pad + s * per_s

        for j in range(nranges // _NC):
            r = c * (nranges // _NC) + j
            r_base = r * nr

            for t in range((nch + _NS - 1) // _NS):
                cid = t * _NS + s

                @pl.when(cid < nch)
                def _():
                    pltpu.sync_copy(
                        y_hbm.at[pl.ds(r_base + cid * lch, lch)],
                        shared.at[pl.ds(cid * lch, lch)])
            plsc.subcore_barrier()

            depth = nbuf // 2

            def issue_load(k, p):
                pltpu.async_copy(
                    opad_hbm.at[pl.ds(base + k * chunk, chunk)],
                    idx_raw.at[p], isem[p])
                pltpu.async_copy(
                    y_hbm.at[pl.ds(base + k * chunk, chunk)],
                    rows[p], lsem[p])

            def drain_add(q):
                pltpu.make_async_copy(
                    rows[q], shared.at[idx_loc.at[q]], asem[q]).wait()

            for b in range(depth):
                issue_load(b, b)

            def ring_body(g, _):
                for b in range(nbuf):
                    i = g * nbuf + b
                    pltpu.make_async_copy(
                        opad_hbm.at[pl.ds(0, chunk)], idx_raw.at[b],
                        isem[b]).wait()
                    for v in range(chunk // _LANES):
                        d = idx_raw[b, pl.ds(v * _LANES, _LANES)]
                        lo = d - r_base
                        ok = (lo >= 0) & (lo < nr)
                        idx_loc[b, pl.ds(v * _LANES, _LANES)] = jnp.where(
                            ok, lo, dump)
                    pltpu.make_async_copy(
                        y_hbm.at[pl.ds(0, chunk)], rows[b],
                        lsem[b]).wait()
                    pltpu.async_copy(rows[b], shared.at[idx_loc.at[b]],
                                     asem[b], add=True)
                    q = (b - depth) % nbuf
                    if b >= depth:
                        drain_add(q)
                    else:
                        @pl.when(g > 0)
                        def _():
                            drain_add(q)

                    @pl.when(i + depth < iters)
                    def _():
                        issue_load(i + depth, q)
                return 0

            lax.fori_loop(0, outer, ring_body, 0)
            for t in range(depth):
                drain_add((iters - depth + t) % nbuf)
            plsc.subcore_barrier()

            for t in range((nch + _NS - 1) // _NS):
                cid = t * _NS + s

                @pl.when(cid < nch)
                def _():
                    pltpu.sync_copy(
                        shared.at[pl.ds(cid * lch, lch)],
                        out_hbm.at[pl.ds(r_base + cid * lch, lch)])
            plsc.subcore_barrier()

    return scatter_k(y, out_pad)


def kernel(x, bn_gamma, bn_beta, W, in_idx, out_idx, kmap_sizes):
    n, f = x.shape
    kvol, _, fout = W.shape
    e = in_idx.shape[0]
    blk = 512
    center = kvol // 2

    nranges = 4
    nr = ((n + nranges * 512 - 1) // (nranges * 512)) * 512
    n_pad = nranges * nr
    rem_max = (e - n) + (kvol - 1) * (blk - 1)
    s_rem = ((rem_max + 2047) // 2048) * 2048
    while (n_pad + s_rem) % 4096:
        s_rem += 2048
    ep = n_pad + s_rem

    nblk = 25
    sums = _bn_stats(x, nblk)
    h = _norm_silu(x, sums, bn_gamma, bn_beta, nblk)

    perm_k = jnp.array([center] + [k for k in range(kvol) if k != center],
                       dtype=jnp.int32)
    sizes = kmap_sizes.astype(jnp.int32)
    csum = jnp.cumsum(sizes)
    cexcl = csum - sizes
    sizes_ord = sizes[perm_k]
    cexcl_ord = cexcl[perm_k]
    padded_ord = jnp.concatenate([
        jnp.array([n_pad], dtype=jnp.int32),
        ((sizes_ord[1:] + blk - 1) // blk) * blk,
    ])
    ostart = jnp.cumsum(padded_ord) - padded_ord
    p = jnp.arange(ep, dtype=jnp.int32)

    ostart_p = jnp.full((ep,), 0, jnp.int32)
    cexcl_p = jnp.broadcast_to(cexcl_ord[0], (ep,))
    size_p = jnp.broadcast_to(sizes_ord[0], (ep,))
    for k in range(1, kvol):
        sel = p >= ostart[k]
        ostart_p = jnp.where(sel, ostart[k], ostart_p)
        cexcl_p = jnp.where(sel, cexcl_ord[k], cexcl_p)
        size_p = jnp.where(sel, sizes_ord[k], size_p)
    rel = p - ostart_p
    edge = rel + cexcl_p
    valid = rel < size_p
    ec = jnp.clip(edge, 0, e - 1)
    in_pad = jnp.where(valid, in_idx[ec], 0).astype(jnp.int32)
    out_pad = jnp.where(valid, out_idx[ec], -1).astype(jnp.int32)
    pb = jnp.arange(ep // blk, dtype=jnp.int32) * blk
    block_k = jnp.broadcast_to(perm_k[0], (ep // blk,))
    for k in range(1, kvol):
        block_k = jnp.where(pb >= ostart[k], perm_k[k], block_k)

    h_src = _sc_gather(h, in_pad, ep)
    y = _grouped_matmul(h_src, W, block_k, blk)
    out = _sc_scatter_add(y, out_pad, n_pad)
    return out[:n]


# speedup vs baseline: 1.2829x; 1.2829x over previous
import functools

import jax
import jax.numpy as jnp
from jax import lax
from jax.experimental import pallas as pl
from jax.experimental.pallas import tpu as pltpu
from jax.experimental.pallas import tpu_sc as plsc

_NC = 2
_NS = 16
_LANES = 16


def _stats_body(x_ref, s_ref):
    @pl.when(pl.program_id(0) == 0)
    def _():
        s_ref[...] = jnp.zeros_like(s_ref)

    xb = x_ref[...]
    s0 = jnp.sum(xb, axis=0)
    s1 = jnp.sum(xb * xb, axis=0)
    s_ref[...] += jnp.stack([s0, s1])


def _bn_stats(x, nblk):
    n, f = x.shape
    rows = n // nblk
    return pl.pallas_call(
        _stats_body,
        grid=(nblk,),
        in_specs=[pl.BlockSpec((rows, f), lambda i: (i, 0))],
        out_specs=pl.BlockSpec((2, f), lambda i: (0, 0)),
        out_shape=jax.ShapeDtypeStruct((2, f), jnp.float32),
    )(x)


def _norm_silu_body(n_rows, x_ref, s_ref, g_ref, b_ref, h_ref):
    s = s_ref[...]
    mean = s[0] / n_rows
    var = s[1] / n_rows - mean * mean
    scale = g_ref[0] * lax.rsqrt(var + 1e-5)
    shift = b_ref[0] - mean * scale
    t = x_ref[...] * scale + shift
    h_ref[...] = t * jax.nn.sigmoid(t)


def _norm_silu(x, sums, gamma, beta, nblk):
    n, f = x.shape
    rows = n // nblk
    return pl.pallas_call(
        functools.partial(_norm_silu_body, float(n)),
        grid=(nblk,),
        in_specs=[
            pl.BlockSpec((rows, f), lambda i: (i, 0)),
            pl.BlockSpec((2, f), lambda i: (0, 0)),
            pl.BlockSpec((1, f), lambda i: (0, 0)),
            pl.BlockSpec((1, f), lambda i: (0, 0)),
        ],
        out_specs=pl.BlockSpec((rows, f), lambda i: (i, 0)),
        out_shape=jax.ShapeDtypeStruct((n, f), jnp.float32),
    )(x, sums, gamma.reshape(1, f), beta.reshape(1, f))


def _sc_gather(h, in_pad, ep):
    n, f = h.shape
    chunk = 128
    nbuf = 6
    per_w = ep // (_NC * _NS)
    iters = per_w // chunk
    mesh = plsc.VectorSubcoreMesh(core_axis_name="c", subcore_axis_name="s")

    @functools.partial(
        pl.kernel,
        mesh=mesh,
        out_type=jax.ShapeDtypeStruct((ep, f), jnp.float32),
        scratch_types=[
            pltpu.VMEM((per_w,), jnp.int32),
            *[pltpu.VMEM((chunk, f), jnp.float32) for _ in range(nbuf)],
            *[pltpu.SemaphoreType.DMA for _ in range(2 * nbuf)],
        ],
    )
    def gather_k(h_hbm, idx_hbm, out_hbm, idx_all, *bufs_sems):
        rows = bufs_sems[:nbuf]
        gsem = bufs_sems[nbuf:2 * nbuf]
        ssem = bufs_sems[2 * nbuf:]
        wid = lax.axis_index("s") * _NC + lax.axis_index("c")
        base = wid * per_w

        pltpu.sync_copy(idx_hbm.at[pl.ds(base, per_w)], idx_all)

        depth = nbuf - 1
        ghandles = [None] * nbuf
        shandles = [None] * nbuf

        def issue_gather(j):
            p = j % nbuf
            ghandles[p] = pltpu.async_copy(
                h_hbm.at[idx_all.at[pl.ds(j * chunk, chunk)]],
                rows[p], gsem[p])

        for j in range(min(depth, iters)):
            issue_gather(j)
        for i in range(iters):
            p = i % nbuf
            ghandles[p].wait()
            shandles[p] = pltpu.async_copy(
                rows[p], out_hbm.at[pl.ds(base + i * chunk, chunk)], ssem[p])
            j = i + depth
            if j < iters:
                pj = j % nbuf
                if shandles[pj] is not None:
                    shandles[pj].wait()
                issue_gather(j)
        for p in range(nbuf):
            if shandles[p] is not None:
                shandles[p].wait()

    return gather_k(h, in_pad)


def _mm_body(bk_ref, x_ref, w_ref, y_ref):
    del bk_ref
    y_ref[...] = jnp.dot(x_ref[...], w_ref[0],
                         preferred_element_type=jnp.float32)


def _grouped_matmul(h_src, w, block_k, blk):
    ep, f = h_src.shape
    fout = w.shape[-1]
    nb = ep // blk
    grid_spec = pltpu.PrefetchScalarGridSpec(
        num_scalar_prefetch=1,
        grid=(nb,),
        in_specs=[
            pl.BlockSpec((blk, f), lambda b, bk: (b, 0)),
            pl.BlockSpec((1, f, fout), lambda b, bk: (bk[b], 0, 0)),
        ],
        out_specs=pl.BlockSpec((blk, fout), lambda b, bk: (b, 0)),
    )
    return pl.pallas_call(
        _mm_body,
        grid_spec=grid_spec,
        out_shape=jax.ShapeDtypeStruct((ep, fout), jnp.float32),
    )(block_k, h_src, w)


def _sc_scatter_add(y, out_pad, n_pad):
    ep, f = y.shape
    chunk = 32
    nranges = 4
    nr = n_pad // nranges
    region = nr + 8
    dump = nr
    lch = 512
    nch = nr // lch
    rem = ep - n_pad
    per_s = rem // _NS
    iters = per_s // chunk
    nbuf = 4
    outer = iters // nbuf
    mesh = plsc.VectorSubcoreMesh(core_axis_name="c", subcore_axis_name="s")

    @functools.partial(
        pl.kernel,
        mesh=mesh,
        out_type=jax.ShapeDtypeStruct((n_pad, f), jnp.float32),
        scratch_types=[
            pltpu.VMEM((nbuf, chunk), jnp.int32),
            pltpu.VMEM((nbuf, chunk), jnp.int32),
            pltpu.VMEM_SHARED((region, f), jnp.float32),
            *[pltpu.VMEM((chunk, f), jnp.float32) for _ in range(nbuf)],
            *[pltpu.SemaphoreType.DMA for _ in range(3 * nbuf)],
        ],
    )
    def scatter_k(y_hbm, opad_hbm, out_hbm, idx_raw, idx_loc, shared,
                  *bufs_sems):
        rows = bufs_sems[:nbuf]
        lsem = bufs_sems[nbuf:2 * nbuf]
        isem = bufs_sems[2 * nbuf:3 * nbuf]
        asem = bufs_sems[3 * nbuf:]
        c = lax.axis_index("c")
        s = lax.axis_index("s")
        rbase = s * per_s
        base = n_pad + rbase

        for j in range(nranges // _NC):
            r = c * (nranges // _NC) + j
            r_base = r * nr

            for t in range((nch + _NS - 1) // _NS):
                cid = t * _NS + s

                @pl.when(cid < nch)
                def _():
                    pltpu.sync_copy(
                        y_hbm.at[pl.ds(r_base + cid * lch, lch)],
                        shared.at[pl.ds(cid * lch, lch)])
            plsc.subcore_barrier()

            depth = nbuf // 2

            def issue_load(k, p):
                pltpu.async_copy(
                    opad_hbm.at[pl.ds(rbase + k * chunk, chunk)],
                    idx_raw.at[p], isem[p])
                pltpu.async_copy(
                    y_hbm.at[pl.ds(base + k * chunk, chunk)],
                    rows[p], lsem[p])

            def drain_add(q):
                pltpu.make_async_copy(
                    rows[q], shared.at[idx_loc.at[q]], asem[q]).wait()

            for b in range(depth):
                issue_load(b, b)

            def ring_body(g, _):
                for b in range(nbuf):
                    i = g * nbuf + b
                    pltpu.make_async_copy(
                        opad_hbm.at[pl.ds(0, chunk)], idx_raw.at[b],
                        isem[b]).wait()
                    for v in range(chunk // _LANES):
                        d = idx_raw[b, pl.ds(v * _LANES, _LANES)]
                        lo = d - r_base
                        ok = (lo >= 0) & (lo < nr)
                        idx_loc[b, pl.ds(v * _LANES, _LANES)] = jnp.where(
                            ok, lo, dump)
                    pltpu.make_async_copy(
                        y_hbm.at[pl.ds(0, chunk)], rows[b],
                        lsem[b]).wait()
                    pltpu.async_copy(rows[b], shared.at[idx_loc.at[b]],
                                     asem[b], add=True)
                    q = (b - depth) % nbuf
                    if b >= depth:
                        drain_add(q)
                    else:
                        @pl.when(g > 0)
                        def _():
                            drain_add(q)

                    @pl.when(i + depth < iters)
                    def _():
                        issue_load(i + depth, q)
                return 0

            lax.fori_loop(0, outer, ring_body, 0)
            for t in range(depth):
                drain_add((iters - depth + t) % nbuf)
            plsc.subcore_barrier()

            for t in range((nch + _NS - 1) // _NS):
                cid = t * _NS + s

                @pl.when(cid < nch)
                def _():
                    pltpu.sync_copy(
                        shared.at[pl.ds(cid * lch, lch)],
                        out_hbm.at[pl.ds(r_base + cid * lch, lch)])
            plsc.subcore_barrier()

    return scatter_k(y, out_pad)


def kernel(x, bn_gamma, bn_beta, W, in_idx, out_idx, kmap_sizes):
    n, f = x.shape
    kvol, _, fout = W.shape
    e = in_idx.shape[0]
    blk = 128
    center = kvol // 2

    nranges = 4
    nr = ((n + nranges * 512 - 1) // (nranges * 512)) * 512
    n_pad = nranges * nr
    rem_max = (e - n) + (kvol - 1) * (blk - 1)
    s_rem = ((rem_max + 2047) // 2048) * 2048
    while (n_pad + s_rem) % 4096:
        s_rem += 2048
    ep = n_pad + s_rem

    nblk = 25
    sums = _bn_stats(x, nblk)
    h = _norm_silu(x, sums, bn_gamma, bn_beta, nblk)

    perm_k = jnp.array([center] + [k for k in range(kvol) if k != center],
                       dtype=jnp.int32)
    sizes = kmap_sizes.astype(jnp.int32)
    csum = jnp.cumsum(sizes)
    cexcl = csum - sizes
    sizes_ord = sizes[perm_k]
    cexcl_ord = cexcl[perm_k]
    padded_ord = jnp.concatenate([
        jnp.array([n_pad], dtype=jnp.int32),
        ((sizes_ord[1:] + blk - 1) // blk) * blk,
    ])
    ostart = jnp.cumsum(padded_ord) - padded_ord

    in_pad_c = jnp.concatenate([
        lax.dynamic_slice(in_idx, (cexcl[center],), (n,)),
        jnp.zeros((n_pad - n,), jnp.int32),
    ])

    p = jnp.arange(s_rem, dtype=jnp.int32) + n_pad
    ostart_p = jnp.broadcast_to(ostart[1], (s_rem,))
    cexcl_p = jnp.broadcast_to(cexcl_ord[1], (s_rem,))
    size_p = jnp.broadcast_to(sizes_ord[1], (s_rem,))
    for k in range(2, kvol):
        sel = p >= ostart[k]
        ostart_p = jnp.where(sel, ostart[k], ostart_p)
        cexcl_p = jnp.where(sel, cexcl_ord[k], cexcl_p)
        size_p = jnp.where(sel, sizes_ord[k], size_p)
    rel = p - ostart_p
    edge = rel + cexcl_p
    valid = rel < size_p
    ec = jnp.clip(edge, 0, e - 1)
    in_pad = jnp.concatenate([
        in_pad_c, jnp.where(valid, in_idx[ec], 0).astype(jnp.int32)])
    out_pad = jnp.where(valid, out_idx[ec], -1).astype(jnp.int32)
    pb = jnp.arange(ep // blk, dtype=jnp.int32) * blk
    block_k = jnp.broadcast_to(perm_k[0], (ep // blk,))
    for k in range(1, kvol):
        block_k = jnp.where(pb >= ostart[k], perm_k[k], block_k)

    h_src = _sc_gather(h, in_pad, ep)
    y = _grouped_matmul(h_src, W, block_k, blk)
    out = _sc_scatter_add(y, out_pad, n_pad)
    return out[:n]


# speedup vs baseline: 1.3120x; 1.0227x over previous
import functools

import jax
import jax.numpy as jnp
from jax import lax
from jax.experimental import pallas as pl
from jax.experimental.pallas import tpu as pltpu
from jax.experimental.pallas import tpu_sc as plsc

_NC = 2
_NS = 16
_LANES = 16


def _stats_body(x_ref, s_ref):
    @pl.when(pl.program_id(0) == 0)
    def _():
        s_ref[...] = jnp.zeros_like(s_ref)

    xb = x_ref[...]
    s0 = jnp.sum(xb, axis=0)
    s1 = jnp.sum(xb * xb, axis=0)
    s_ref[...] += jnp.stack([s0, s1])


def _bn_stats(x, nblk):
    n, f = x.shape
    rows = n // nblk
    return pl.pallas_call(
        _stats_body,
        grid=(nblk,),
        in_specs=[pl.BlockSpec((rows, f), lambda i: (i, 0))],
        out_specs=pl.BlockSpec((2, f), lambda i: (0, 0)),
        out_shape=jax.ShapeDtypeStruct((2, f), jnp.float32),
    )(x)


def _norm_silu_body(n_rows, x_ref, s_ref, g_ref, b_ref, h_ref):
    s = s_ref[...]
    mean = s[0] / n_rows
    var = s[1] / n_rows - mean * mean
    scale = g_ref[0] * lax.rsqrt(var + 1e-5)
    shift = b_ref[0] - mean * scale
    t = x_ref[...] * scale + shift
    h_ref[...] = t * jax.nn.sigmoid(t)


def _norm_silu(x, sums, gamma, beta, nblk):
    n, f = x.shape
    rows = n // nblk
    return pl.pallas_call(
        functools.partial(_norm_silu_body, float(n)),
        grid=(nblk,),
        in_specs=[
            pl.BlockSpec((rows, f), lambda i: (i, 0)),
            pl.BlockSpec((2, f), lambda i: (0, 0)),
            pl.BlockSpec((1, f), lambda i: (0, 0)),
            pl.BlockSpec((1, f), lambda i: (0, 0)),
        ],
        out_specs=pl.BlockSpec((rows, f), lambda i: (i, 0)),
        out_shape=jax.ShapeDtypeStruct((n, f), jnp.float32),
    )(x, sums, gamma.reshape(1, f), beta.reshape(1, f))


def _sc_gather(h, in_pad, ep):
    n, f = h.shape
    chunk = 128
    nbuf = 6
    per_w = ep // (_NC * _NS)
    iters = per_w // chunk
    mesh = plsc.VectorSubcoreMesh(core_axis_name="c", subcore_axis_name="s")

    @functools.partial(
        pl.kernel,
        mesh=mesh,
        out_type=jax.ShapeDtypeStruct((ep, f), jnp.float32),
        scratch_types=[
            pltpu.VMEM((per_w,), jnp.int32),
            *[pltpu.VMEM((chunk, f), jnp.float32) for _ in range(nbuf)],
            *[pltpu.SemaphoreType.DMA for _ in range(2 * nbuf)],
        ],
    )
    def gather_k(h_hbm, idx_hbm, out_hbm, idx_all, *bufs_sems):
        rows = bufs_sems[:nbuf]
        gsem = bufs_sems[nbuf:2 * nbuf]
        ssem = bufs_sems[2 * nbuf:]
        wid = lax.axis_index("s") * _NC + lax.axis_index("c")
        base = wid * per_w

        pltpu.sync_copy(idx_hbm.at[pl.ds(base, per_w)], idx_all)

        depth = nbuf - 1
        ghandles = [None] * nbuf
        shandles = [None] * nbuf

        def issue_gather(j):
            p = j % nbuf
            ghandles[p] = pltpu.async_copy(
                h_hbm.at[idx_all.at[pl.ds(j * chunk, chunk)]],
                rows[p], gsem[p])

        for j in range(min(depth, iters)):
            issue_gather(j)
        for i in range(iters):
            p = i % nbuf
            ghandles[p].wait()
            shandles[p] = pltpu.async_copy(
                rows[p], out_hbm.at[pl.ds(base + i * chunk, chunk)], ssem[p])
            j = i + depth
            if j < iters:
                pj = j % nbuf
                if shandles[pj] is not None:
                    shandles[pj].wait()
                issue_gather(j)
        for p in range(nbuf):
            if shandles[p] is not None:
                shandles[p].wait()

    return gather_k(h, in_pad)


def _mm_body(bk_ref, x_ref, w_ref, y_ref):
    del bk_ref
    y_ref[...] = jnp.dot(x_ref[...], w_ref[0],
                         preferred_element_type=jnp.float32)


def _grouped_matmul(h_src, w, block_k, blk):
    ep, f = h_src.shape
    fout = w.shape[-1]
    nb = ep // blk
    grid_spec = pltpu.PrefetchScalarGridSpec(
        num_scalar_prefetch=1,
        grid=(nb,),
        in_specs=[
            pl.BlockSpec((blk, f), lambda b, bk: (b, 0)),
            pl.BlockSpec((1, f, fout), lambda b, bk: (bk[b], 0, 0)),
        ],
        out_specs=pl.BlockSpec((blk, fout), lambda b, bk: (b, 0)),
    )
    return pl.pallas_call(
        _mm_body,
        grid_spec=grid_spec,
        out_shape=jax.ShapeDtypeStruct((ep, fout), jnp.float32),
    )(block_k, h_src, w)


def _sc_scatter_add(y, out_pad, n_pad):
    ep, f = y.shape
    chunk = 32
    nranges = 4
    nr = n_pad // nranges
    region = nr + 8
    dump = nr
    lch = 512
    nch = nr // lch
    rem = ep - n_pad
    per_s = rem // _NS
    iters = per_s // chunk
    nbuf = 4
    outer = iters // nbuf
    mesh = plsc.VectorSubcoreMesh(core_axis_name="c", subcore_axis_name="s")

    @functools.partial(
        pl.kernel,
        mesh=mesh,
        out_type=jax.ShapeDtypeStruct((n_pad, f), jnp.float32),
        scratch_types=[
            pltpu.VMEM((nbuf, chunk), jnp.int32),
            pltpu.VMEM((nbuf, chunk), jnp.int32),
            pltpu.VMEM_SHARED((region, f), jnp.float32),
            *[pltpu.VMEM((chunk, f), jnp.float32) for _ in range(nbuf)],
            *[pltpu.SemaphoreType.DMA for _ in range(3 * nbuf)],
        ],
    )
    def scatter_k(y_hbm, opad_hbm, out_hbm, idx_raw, idx_loc, shared,
                  *bufs_sems):
        rows = bufs_sems[:nbuf]
        lsem = bufs_sems[nbuf:2 * nbuf]
        isem = bufs_sems[2 * nbuf:3 * nbuf]
        asem = bufs_sems[3 * nbuf:]
        c = lax.axis_index("c")
        s = lax.axis_index("s")
        base = n_pad + s * per_s

        for j in range(nranges // _NC):
            r = c * (nranges // _NC) + j
            r_base = r * nr

            for t in range((nch + _NS - 1) // _NS):
                cid = t * _NS + s

                @pl.when(cid < nch)
                def _():
                    pltpu.sync_copy(
                        y_hbm.at[pl.ds(r_base + cid * lch, lch)],
                        shared.at[pl.ds(cid * lch, lch)])
            plsc.subcore_barrier()

            depth = nbuf // 2

            def issue_load(k, p):
                pltpu.async_copy(
                    opad_hbm.at[pl.ds(base + k * chunk, chunk)],
                    idx_raw.at[p], isem[p])
                pltpu.async_copy(
                    y_hbm.at[pl.ds(base + k * chunk, chunk)],
                    rows[p], lsem[p])

            def drain_add(q):
                pltpu.make_async_copy(
                    rows[q], shared.at[idx_loc.at[q]], asem[q]).wait()

            for b in range(depth):
                issue_load(b, b)

            def ring_body(g, _):
                for b in range(nbuf):
                    i = g * nbuf + b
                    pltpu.make_async_copy(
                        opad_hbm.at[pl.ds(0, chunk)], idx_raw.at[b],
                        isem[b]).wait()
                    for v in range(chunk // _LANES):
                        d = idx_raw[b, pl.ds(v * _LANES, _LANES)]
                        lo = d - r_base
                        ok = (lo >= 0) & (lo < nr)
                        idx_loc[b, pl.ds(v * _LANES, _LANES)] = jnp.where(
                            ok, lo, dump)
                    pltpu.make_async_copy(
                        y_hbm.at[pl.ds(0, chunk)], rows[b],
                        lsem[b]).wait()
                    pltpu.async_copy(rows[b], shared.at[idx_loc.at[b]],
                                     asem[b], add=True)
                    q = (b - depth) % nbuf
                    if b >= depth:
                        drain_add(q)
                    else:
                        @pl.when(g > 0)
                        def _():
                            drain_add(q)

                    @pl.when(i + depth < iters)
                    def _():
                        issue_load(i + depth, q)
                return 0

            lax.fori_loop(0, outer, ring_body, 0)
            for t in range(depth):
                drain_add((iters - depth + t) % nbuf)
            plsc.subcore_barrier()

            for t in range((nch + _NS - 1) // _NS):
                cid = t * _NS + s

                @pl.when(cid < nch)
                def _():
                    pltpu.sync_copy(
                        shared.at[pl.ds(cid * lch, lch)],
                        out_hbm.at[pl.ds(r_base + cid * lch, lch)])
            plsc.subcore_barrier()

    return scatter_k(y, out_pad)


def kernel(x, bn_gamma, bn_beta, W, in_idx, out_idx, kmap_sizes):
    n, f = x.shape
    kvol, _, fout = W.shape
    e = in_idx.shape[0]
    blk = 128
    center = kvol // 2

    nranges = 4
    nr = ((n + nranges * 512 - 1) // (nranges * 512)) * 512
    n_pad = nranges * nr
    rem_max = (e - n) + (kvol - 1) * (blk - 1)
    s_rem = ((rem_max + 2047) // 2048) * 2048
    while (n_pad + s_rem) % 4096:
        s_rem += 2048
    ep = n_pad + s_rem

    nblk = 25
    sums = _bn_stats(x, nblk)
    h = _norm_silu(x, sums, bn_gamma, bn_beta, nblk)

    perm_k = jnp.array([center] + [k for k in range(kvol) if k != center],
                       dtype=jnp.int32)
    sizes = kmap_sizes.astype(jnp.int32)
    csum = jnp.cumsum(sizes)
    cexcl = csum - sizes
    sizes_ord = sizes[perm_k]
    cexcl_ord = cexcl[perm_k]
    padded_ord = jnp.concatenate([
        jnp.array([n_pad], dtype=jnp.int32),
        ((sizes_ord[1:] + blk - 1) // blk) * blk,
    ])
    ostart = jnp.cumsum(padded_ord) - padded_ord
    p = jnp.arange(ep, dtype=jnp.int32)

    ostart_p = jnp.full((ep,), 0, jnp.int32)
    cexcl_p = jnp.broadcast_to(cexcl_ord[0], (ep,))
    size_p = jnp.broadcast_to(sizes_ord[0], (ep,))
    for k in range(1, kvol):
        sel = p >= ostart[k]
        ostart_p = jnp.where(sel, ostart[k], ostart_p)
        cexcl_p = jnp.where(sel, cexcl_ord[k], cexcl_p)
        size_p = jnp.where(sel, sizes_ord[k], size_p)
    rel = p - ostart_p
    edge = rel + cexcl_p
    valid = rel < size_p
    ec = jnp.clip(edge, 0, e - 1)
    in_pad = jnp.where(valid, in_idx[ec], 0).astype(jnp.int32)
    out_pad = jnp.where(valid, out_idx[ec], -1).astype(jnp.int32)
    pb = jnp.arange(ep // blk, dtype=jnp.int32) * blk
    block_k = jnp.broadcast_to(perm_k[0], (ep // blk,))
    for k in range(1, kvol):
        block_k = jnp.where(pb >= ostart[k], perm_k[k], block_k)

    h_src = _sc_gather(h, in_pad, ep)
    y = _grouped_matmul(h_src, W, block_k, blk)
    out = _sc_scatter_add(y, out_pad, n_pad)
    return out[:n]
